# Initial kernel scaffold; baseline (speedup 1.0000x reference)
import jax
import jax.numpy as jnp
from jax.experimental import pallas as pl


def kernel(x, edge_index, weight):
    raise NotImplementedError("write your pallas kernel here")


# speedup vs baseline: 1.9158x; 1.9158x over previous
import jax
import jax.numpy as jnp
from jax.experimental import pallas as pl


def _get_score_kernel(x_ref, w_ref, xout_ref, score_ref):
    xv = x_ref[...]
    w = w_ref[...]
    s_row = jax.lax.dot_general(
        w, xv, (((1,), (1,)), ((), ())), preferred_element_type=jnp.float32
    )
    s_col = jax.lax.dot_general(
        xv, w, (((1,), (1,)), ((), ())), preferred_element_type=jnp.float32
    )
    m = jnp.mean(s_row)
    inv_norm = jax.lax.rsqrt(jnp.sum(w * w))
    sc_row = jnp.tanh((s_row - m) * inv_norm)
    sc_col = jnp.tanh((s_col - m) * inv_norm)
    xout_ref[...] = xv * sc_col
    score_ref[...] = sc_row


def kernel(x, edge_index, weight):
    n, d = x.shape
    x_out, score = pl.pallas_call(
        _get_score_kernel,
        out_shape=(
            jax.ShapeDtypeStruct((n, d), x.dtype),
            jax.ShapeDtypeStruct((1, n), x.dtype),
        ),
    )(x, weight)
    return x_out, score
